# Initial kernel scaffold; baseline (speedup 1.0000x reference)
import jax
import jax.numpy as jnp
from jax.experimental import pallas as pl


def kernel(input_session, support_nodes_layer1, support_nodes_layer2, support_sessions_layer1, support_sessions_layer2, edge_index1, edge_index2, user_emb, item_emb, Wih, Whh, bih, bhh, W1, fc1_w, fc1_b, fc2_w, fc2_b, W2):
    raise NotImplementedError("write your pallas kernel here")


# speedup vs baseline: 1.0588x; 1.0588x over previous
import functools

import jax
import jax.numpy as jnp
from jax import lax
from jax.experimental import pallas as pl
from jax.experimental.pallas import tpu as pltpu
from jax.experimental.pallas import tpu_sc as plsc

D = 50
B = 1024
S2 = 5
E = 16384
T = 20
NV = 99999

NC, NS = 2, 16
NW = NC * NS
ROWS_W = (B * T) // NW
CHUNK = 128
NCH = ROWS_W // CHUNK
EW = E // NW
HBIN = S2 * B
NB2 = 8 * B

CB = 2048


def _sc_body(item_hbm, user_hbm, idx_flat, src_flat, dst_flat, iu8, ii8,
             emb_out, cnt_out, ur_out, ir_out,
             idx_f, rows_v, sv, dv, hist, hist2, i8v, r8v, sem):
    wid = lax.axis_index("s") * NC + lax.axis_index("c")
    lane = lax.iota(jnp.int32, 16)
    zeros16 = jnp.zeros((16,), jnp.float32)
    ones16 = jnp.ones((16,), jnp.float32)

    pltpu.sync_copy(idx_flat.at[pl.ds(wid * ROWS_W, ROWS_W)], idx_f)
    for j in range(NCH):
        pltpu.async_copy(item_hbm.at[idx_f.at[pl.ds(j * CHUNK, CHUNK)]],
                         rows_v.at[pl.ds(j * CHUNK, CHUNK)], sem).wait()

    def _zero_body(i, carry):
        idx16 = idx_f[pl.ds(i * 16, 16)]
        for l in range(16):
            @pl.when(idx16[l] == 0)
            def _():
                r = i * 16 + l
                for off in (0, 16, 32, 34):
                    rows_v[r, pl.ds(off, 16)] = zeros16
        return carry
    lax.fori_loop(0, ROWS_W // 16, _zero_body, 0)
    pltpu.sync_copy(rows_v, emb_out.at[pl.ds(wid * ROWS_W, ROWS_W)])

    def _hz(i, carry):
        for u in range(8):
            hist[pl.ds(i * 128 + u * 16, 16)] = zeros16
        return carry
    lax.fori_loop(0, (16 * HBIN) // 128, _hz, 0)

    def _hz2(i, carry):
        for u in range(8):
            hist2[pl.ds(i * 128 + u * 16, 16)] = zeros16
        return carry
    lax.fori_loop(0, NB2 // 128, _hz2, 0)

    pltpu.sync_copy(src_flat.at[pl.ds(wid * EW, EW)], sv)
    pltpu.sync_copy(dst_flat.at[pl.ds(wid * EW, EW)], dv)

    def _acc(i, carry):
        s16 = sv[pl.ds(i * 16, 16)]
        d16 = dv[pl.ds(i * 16, 16)]
        k = lane * HBIN + d16 * S2 + s16
        plsc.addupdate_scatter(hist, [k], ones16, mask=lane < 16)
        return carry
    lax.fori_loop(0, EW // 16, _acc, 0)

    def _red(c, carry):
        base = c * 16
        acc = hist[pl.ds(base, 16)]
        for l in range(1, 16):
            acc = acc + hist[pl.ds(l * HBIN + base, 16)]
        j = base + lane
        bvec = j // S2
        svec = j - bvec * S2
        plsc.store_scatter(hist2, [svec * B + bvec], acc)
        return carry
    lax.fori_loop(0, HBIN // 16, _red, 0)
    pltpu.sync_copy(hist2, cnt_out.at[pl.ds(wid * NB2, NB2)])

    @pl.when(wid == 0)
    def _():
        pltpu.sync_copy(iu8, i8v)
        pltpu.async_copy(user_hbm.at[i8v], r8v, sem).wait()
        pltpu.sync_copy(r8v, ur_out)
        pltpu.sync_copy(ii8, i8v)
        pltpu.async_copy(item_hbm.at[i8v], r8v, sem).wait()
        pltpu.sync_copy(r8v, ir_out)


@functools.cache
def _make_sc_call():
  return pl.kernel(
    _sc_body,
    out_type=(
        jax.ShapeDtypeStruct((T * B, D), jnp.float32),
        jax.ShapeDtypeStruct((NW * NB2,), jnp.float32),
        jax.ShapeDtypeStruct((8, D), jnp.float32),
        jax.ShapeDtypeStruct((8, D), jnp.float32),
    ),
    mesh=plsc.VectorSubcoreMesh(core_axis_name="c", subcore_axis_name="s",
                                num_cores=NC, num_subcores=NS),
    compiler_params=pltpu.CompilerParams(needs_layout_passes=False,
                                         use_tc_tiling_on_sc=False),
    scratch_types=[
        pltpu.VMEM((ROWS_W,), jnp.int32),
        pltpu.VMEM((ROWS_W, D), jnp.float32),
        pltpu.VMEM((EW,), jnp.int32),
        pltpu.VMEM((EW,), jnp.int32),
        pltpu.VMEM((16 * HBIN,), jnp.float32),
        pltpu.VMEM((NB2,), jnp.float32),
        pltpu.VMEM((8,), jnp.int32),
        pltpu.VMEM((8, D), jnp.float32),
        pltpu.SemaphoreType.DMA,
    ],
  )


def _a_body(emb, cnts, ur, ir, s2c, Wx, Wh, bg, W1a, W1b, f2w, f2b, W2a, W2b,
            sr_out):
    f32 = jnp.float32
    bias = bg[0:1, :]

    def step(t, hc):
        h, c = hc
        x = emb[pl.ds(t * B, B), :]
        g = (jnp.dot(x, Wx[:], preferred_element_type=f32)
             + jnp.dot(h, Wh[:], preferred_element_type=f32) + bias)
        i = jax.nn.sigmoid(g[:, 0:128])
        f = jax.nn.sigmoid(g[:, 128:256])
        gg = jnp.tanh(g[:, 256:384])
        o = jax.nn.sigmoid(g[:, 384:512])
        c = f * c + i * gg
        h = o * jnp.tanh(c)
        return (h, c)

    h0 = jnp.zeros((B, 128), f32)
    h, _ = lax.fori_loop(0, T, step, (h0, h0))

    x5 = ir[:] * (s2c[:] != 0).astype(f32)
    g5 = jnp.dot(x5, Wx[:], preferred_element_type=f32) + bias
    c5 = jax.nn.sigmoid(g5[:, 0:128]) * jnp.tanh(g5[:, 256:384])
    h5 = jax.nn.sigmoid(g5[:, 384:512]) * jnp.tanh(c5)

    ls2 = jnp.maximum(jnp.dot(ur[:], W1a[:], preferred_element_type=f32)
                      + jnp.dot(h5, W1b[:], preferred_element_type=f32), 0.0)
    rowm = (lax.broadcasted_iota(jnp.int32, (8, 1), 0) < S2).astype(f32)
    ls2 = ls2 * rowm

    cbs = jnp.sum(cnts[:], axis=0).T
    P = lax.dot_general(h, ls2, (((1,), (1,)), ((), ())),
                        preferred_element_type=f32)
    mx = jnp.max(P, axis=1, keepdims=True)
    wt = cbs * jnp.exp(P - mx)
    den = jnp.sum(wt, axis=1, keepdims=True)
    wn = wt / jnp.where(den == 0.0, 1.0, den)
    rst = jnp.dot(wn, ls2, preferred_element_type=f32)
    feat = jnp.maximum(jnp.dot(rst, f2w[:], preferred_element_type=f32)
                       + f2b[0:1, :], 0.0)
    sr_out[:] = (jnp.dot(h, W2a[:], preferred_element_type=f32)
                 + jnp.dot(feat, W2b[:], preferred_element_type=f32))


def _c_body(sr_ref, it_ref, out_ref):
    out_ref[:] = jnp.dot(sr_ref[:], it_ref[:], preferred_element_type=jnp.float32)


def kernel(input_session, support_nodes_layer1, support_nodes_layer2,
           support_sessions_layer1, support_sessions_layer2,
           edge_index1, edge_index2,
           user_emb, item_emb, Wih, Whh, bih, bhh,
           W1, fc1_w, fc1_b, fc2_w, fc2_b, W2):
    f32 = jnp.float32
    i32 = jnp.int32

    idx_flat = input_session.T.reshape(-1).astype(i32)
    src_flat = edge_index2[0].astype(i32)
    dst_flat = edge_index2[1].astype(i32)
    iu8 = jnp.zeros((8,), i32).at[:S2].set(support_nodes_layer2.astype(i32))
    s2first = support_sessions_layer2[:, 0].astype(i32)
    ii8 = jnp.zeros((8,), i32).at[:S2].set(s2first)
    s2c = jnp.zeros((8, 1), i32).at[:S2, 0].set(s2first)

    def padto(a, r, c):
        return jnp.zeros((r, c), f32).at[:a.shape[0], :a.shape[1]].set(a)

    WihT, WhhT = Wih.T, Whh.T
    Wx = jnp.zeros((D, 512), f32)
    Wh = jnp.zeros((128, 512), f32)
    bg = jnp.zeros((8, 512), f32)
    bsum = bih + bhh
    for g in range(4):
        Wx = Wx.at[:, g * 128:g * 128 + D].set(WihT[:, g * D:(g + 1) * D])
        Wh = Wh.at[:D, g * 128:g * 128 + D].set(WhhT[:, g * D:(g + 1) * D])
        bg = bg.at[:, g * 128:g * 128 + D].set(
            jnp.broadcast_to(bsum[g * D:(g + 1) * D], (8, D)))
    W1a = padto(W1[:, :D].T, D, 128)
    W1b = padto(W1[:, D:].T, 128, 128)
    f2w = padto(fc2_w.T, 128, 128)
    f2b = jnp.zeros((8, 128), f32).at[:, :D].set(jnp.broadcast_to(fc2_b, (8, D)))
    W2a = padto(W2[:, :D].T, 128, D)
    W2b = padto(W2[:, D:].T, 128, D)
    item_T = item_emb[1:].T

    emb, cnts, ur, ir = _make_sc_call()(item_emb, user_emb, idx_flat,
                                        src_flat, dst_flat, iu8, ii8)
    cnts3 = cnts.reshape(NW, 8, B)

    sr = pl.pallas_call(
        _a_body,
        out_shape=jax.ShapeDtypeStruct((B, D), f32),
    )(emb, cnts3, ur, ir, s2c, Wx, Wh, bg, W1a, W1b, f2w, f2b, W2a, W2b)

    nblk = (NV + CB - 1) // CB
    logits = pl.pallas_call(
        _c_body,
        grid=(nblk,),
        in_specs=[pl.BlockSpec((B, D), lambda j: (0, 0)),
                  pl.BlockSpec((D, CB), lambda j: (0, j))],
        out_specs=pl.BlockSpec((B, CB), lambda j: (0, j)),
        out_shape=jax.ShapeDtypeStruct((B, NV), f32),
    )(sr, item_T)
    return logits


# speedup vs baseline: 1.0667x; 1.0074x over previous
import functools

import jax
import jax.numpy as jnp
from jax import lax
from jax.experimental import pallas as pl
from jax.experimental.pallas import tpu as pltpu
from jax.experimental.pallas import tpu_sc as plsc

D = 50
B = 1024
S2 = 5
E = 16384
T = 20
NV = 99999

NC, NS = 2, 16
NW = NC * NS
ROWS_W = (B * T) // NW
CHUNK = 128
NCH = ROWS_W // CHUNK
EW = E // NW
HBIN = S2 * B
NB2 = 8 * B

CB = 2048


def _sc_body(item_hbm, user_hbm, idx_flat, src_flat, dst_flat, iu8, ii8,
             emb_out, cnt_out, ur_out, ir_out,
             idx_f, rows_v, sv, dv, hist, hist2, i8v, r8v, sem):
    wid = lax.axis_index("s") * NC + lax.axis_index("c")
    lane = lax.iota(jnp.int32, 16)
    zeros16 = jnp.zeros((16,), jnp.float32)
    ones16 = jnp.ones((16,), jnp.float32)

    pltpu.sync_copy(idx_flat.at[pl.ds(wid * ROWS_W, ROWS_W)], idx_f)
    for j in range(NCH):
        pltpu.async_copy(item_hbm.at[idx_f.at[pl.ds(j * CHUNK, CHUNK)]],
                         rows_v.at[pl.ds(j * CHUNK, CHUNK)], sem).wait()

    def _zero_body(i, carry):
        idx16 = idx_f[pl.ds(i * 16, 16)]
        for l in range(16):
            @pl.when(idx16[l] == 0)
            def _():
                r = i * 16 + l
                for off in (0, 16, 32, 34):
                    rows_v[r, pl.ds(off, 16)] = zeros16
        return carry
    lax.fori_loop(0, ROWS_W // 16, _zero_body, 0)
    pltpu.sync_copy(rows_v, emb_out.at[pl.ds(wid * ROWS_W, ROWS_W)])

    def _hz(i, carry):
        for u in range(8):
            hist[pl.ds(i * 128 + u * 16, 16)] = zeros16
        return carry
    lax.fori_loop(0, (16 * HBIN) // 128, _hz, 0)

    def _hz2(i, carry):
        for u in range(8):
            hist2[pl.ds(i * 128 + u * 16, 16)] = zeros16
        return carry
    lax.fori_loop(0, NB2 // 128, _hz2, 0)

    pltpu.sync_copy(src_flat.at[pl.ds(wid * EW, EW)], sv)
    pltpu.sync_copy(dst_flat.at[pl.ds(wid * EW, EW)], dv)

    def _acc(i, carry):
        s16 = sv[pl.ds(i * 16, 16)]
        d16 = dv[pl.ds(i * 16, 16)]
        k = lane * HBIN + d16 * S2 + s16
        plsc.addupdate_scatter(hist, [k], ones16, mask=lane < 16)
        return carry
    lax.fori_loop(0, EW // 16, _acc, 0)

    def _red(c, carry):
        base = c * 16
        acc = hist[pl.ds(base, 16)]
        for l in range(1, 16):
            acc = acc + hist[pl.ds(l * HBIN + base, 16)]
        j = base + lane
        bvec = j // S2
        svec = j - bvec * S2
        plsc.store_scatter(hist2, [svec * B + bvec], acc)
        return carry
    lax.fori_loop(0, HBIN // 16, _red, 0)
    pltpu.sync_copy(hist2, cnt_out.at[pl.ds(wid * NB2, NB2)])

    @pl.when(wid == 0)
    def _():
        pltpu.sync_copy(iu8, i8v)
        pltpu.async_copy(user_hbm.at[i8v], r8v, sem).wait()
        pltpu.sync_copy(r8v, ur_out)
        pltpu.sync_copy(ii8, i8v)
        pltpu.async_copy(item_hbm.at[i8v], r8v, sem).wait()
        pltpu.sync_copy(r8v, ir_out)


@functools.cache
def _make_sc_call():
  return pl.kernel(
    _sc_body,
    out_type=(
        jax.ShapeDtypeStruct((T * B, D), jnp.float32),
        jax.ShapeDtypeStruct((NW * NB2,), jnp.float32),
        jax.ShapeDtypeStruct((8, D), jnp.float32),
        jax.ShapeDtypeStruct((8, D), jnp.float32),
    ),
    mesh=plsc.VectorSubcoreMesh(core_axis_name="c", subcore_axis_name="s",
                                num_cores=NC, num_subcores=NS),
    compiler_params=pltpu.CompilerParams(needs_layout_passes=False,
                                         use_tc_tiling_on_sc=False),
    scratch_types=[
        pltpu.VMEM((ROWS_W,), jnp.int32),
        pltpu.VMEM((ROWS_W, D), jnp.float32),
        pltpu.VMEM((EW,), jnp.int32),
        pltpu.VMEM((EW,), jnp.int32),
        pltpu.VMEM((16 * HBIN,), jnp.float32),
        pltpu.VMEM((NB2,), jnp.float32),
        pltpu.VMEM((8,), jnp.int32),
        pltpu.VMEM((8, D), jnp.float32),
        pltpu.SemaphoreType.DMA,
    ],
  )


def _a_body(emb, cnts, ur, ir, s2c, Wx, Wh, bg, W1a, W1b, f2w, f2b, W2a, W2b,
            sr_out):
    f32 = jnp.float32
    bias = bg[0:1, :]

    def step(t, hc):
        h, c = hc
        x = emb[pl.ds(t * B, B), :]
        g = (jnp.dot(x, Wx[:], preferred_element_type=f32)
             + jnp.dot(h, Wh[:], preferred_element_type=f32) + bias)
        i = jax.nn.sigmoid(g[:, 0:128])
        f = jax.nn.sigmoid(g[:, 128:256])
        gg = jnp.tanh(g[:, 256:384])
        o = jax.nn.sigmoid(g[:, 384:512])
        c = f * c + i * gg
        h = o * jnp.tanh(c)
        return (h, c)

    h0 = jnp.zeros((B, 128), f32)
    h, _ = lax.fori_loop(0, T, step, (h0, h0))

    x5 = ir[:] * (s2c[:] != 0).astype(f32)
    g5 = jnp.dot(x5, Wx[:], preferred_element_type=f32) + bias
    c5 = jax.nn.sigmoid(g5[:, 0:128]) * jnp.tanh(g5[:, 256:384])
    h5 = jax.nn.sigmoid(g5[:, 384:512]) * jnp.tanh(c5)

    ls2 = jnp.maximum(jnp.dot(ur[:], W1a[:], preferred_element_type=f32)
                      + jnp.dot(h5, W1b[:], preferred_element_type=f32), 0.0)
    rowm = (lax.broadcasted_iota(jnp.int32, (8, 1), 0) < S2).astype(f32)
    ls2 = ls2 * rowm

    cbs = jnp.sum(cnts[:], axis=0).T
    P = lax.dot_general(h, ls2, (((1,), (1,)), ((), ())),
                        preferred_element_type=f32)
    mx = jnp.max(P, axis=1, keepdims=True)
    wt = cbs * jnp.exp(P - mx)
    den = jnp.sum(wt, axis=1, keepdims=True)
    wn = wt / jnp.where(den == 0.0, 1.0, den)
    rst = jnp.dot(wn, ls2, preferred_element_type=f32)
    feat = jnp.maximum(jnp.dot(rst, f2w[:], preferred_element_type=f32)
                       + f2b[0:1, :], 0.0)
    sr_out[:] = (jnp.dot(h, W2a[:], preferred_element_type=f32)
                 + jnp.dot(feat, W2b[:], preferred_element_type=f32))


def _c_body(sr_ref, ita_ref, itb_ref, out_ref):
    it = jnp.concatenate([ita_ref[1:], itb_ref[0:1]], axis=0)
    out_ref[:] = lax.dot_general(sr_ref[:], it, (((1,), (1,)), ((), ())),
                                 preferred_element_type=jnp.float32)


def kernel(input_session, support_nodes_layer1, support_nodes_layer2,
           support_sessions_layer1, support_sessions_layer2,
           edge_index1, edge_index2,
           user_emb, item_emb, Wih, Whh, bih, bhh,
           W1, fc1_w, fc1_b, fc2_w, fc2_b, W2):
    f32 = jnp.float32
    i32 = jnp.int32

    idx_flat = input_session.T.reshape(-1).astype(i32)
    src_flat = edge_index2[0].astype(i32)
    dst_flat = edge_index2[1].astype(i32)
    iu8 = jnp.zeros((8,), i32).at[:S2].set(support_nodes_layer2.astype(i32))
    s2first = support_sessions_layer2[:, 0].astype(i32)
    ii8 = jnp.zeros((8,), i32).at[:S2].set(s2first)
    s2c = jnp.zeros((8, 1), i32).at[:S2, 0].set(s2first)

    def padto(a, r, c):
        return jnp.zeros((r, c), f32).at[:a.shape[0], :a.shape[1]].set(a)

    WihT, WhhT = Wih.T, Whh.T
    Wx = jnp.zeros((D, 512), f32)
    Wh = jnp.zeros((128, 512), f32)
    bg = jnp.zeros((8, 512), f32)
    bsum = bih + bhh
    for g in range(4):
        Wx = Wx.at[:, g * 128:g * 128 + D].set(WihT[:, g * D:(g + 1) * D])
        Wh = Wh.at[:D, g * 128:g * 128 + D].set(WhhT[:, g * D:(g + 1) * D])
        bg = bg.at[:, g * 128:g * 128 + D].set(
            jnp.broadcast_to(bsum[g * D:(g + 1) * D], (8, D)))
    W1a = padto(W1[:, :D].T, D, 128)
    W1b = padto(W1[:, D:].T, 128, 128)
    f2w = padto(fc2_w.T, 128, 128)
    f2b = jnp.zeros((8, 128), f32).at[:, :D].set(jnp.broadcast_to(fc2_b, (8, D)))
    W2a = padto(W2[:, :D].T, 128, D)
    W2b = padto(W2[:, D:].T, 128, D)

    emb, cnts, ur, ir = _make_sc_call()(item_emb, user_emb, idx_flat,
                                        src_flat, dst_flat, iu8, ii8)
    cnts3 = cnts.reshape(NW, 8, B)

    sr = pl.pallas_call(
        _a_body,
        out_shape=jax.ShapeDtypeStruct((B, D), f32),
    )(emb, cnts3, ur, ir, s2c, Wx, Wh, bg, W1a, W1b, f2w, f2b, W2a, W2b)

    nblk = (NV + CB - 1) // CB
    nb8 = item_emb.shape[0] // 8 - 1
    logits = pl.pallas_call(
        _c_body,
        grid=(nblk,),
        in_specs=[pl.BlockSpec((B, D), lambda j: (0, 0)),
                  pl.BlockSpec((CB, D), lambda j: (j, 0)),
                  pl.BlockSpec((8, D),
                               lambda j: (jnp.minimum((j + 1) * (CB // 8), nb8), 0))],
        out_specs=pl.BlockSpec((B, CB), lambda j: (0, j)),
        out_shape=jax.ShapeDtypeStruct((B, NV), f32),
    )(sr, item_emb, item_emb)
    return logits


# speedup vs baseline: 1.1515x; 1.0795x over previous
import functools

import jax
import jax.numpy as jnp
from jax import lax
from jax.experimental import pallas as pl
from jax.experimental.pallas import tpu as pltpu
from jax.experimental.pallas import tpu_sc as plsc

D = 50
B = 1024
S2 = 5
E = 16384
T = 20
NV = 99999

NC, NS = 2, 16
NW = NC * NS
ROWS_W = (B * T) // NW
CHUNK = 128
NCH = ROWS_W // CHUNK
EW = E // NW
HBIN = S2 * B
NB2 = 8 * B

CB = 4096


def _sc_body(item_hbm, idx_flat, src_flat, dst_flat,
             emb_out, cnt_out,
             idx_f, rows_v, sv, dv, hist, hist2, sem):
    wid = lax.axis_index("s") * NC + lax.axis_index("c")
    lane = lax.iota(jnp.int32, 16)
    zeros16 = jnp.zeros((16,), jnp.float32)
    ones16 = jnp.ones((16,), jnp.float32)

    pltpu.sync_copy(idx_flat.at[pl.ds(wid * ROWS_W, ROWS_W)], idx_f)
    for j in range(NCH):
        pltpu.async_copy(item_hbm.at[idx_f.at[pl.ds(j * CHUNK, CHUNK)]],
                         rows_v.at[pl.ds(j * CHUNK, CHUNK)], sem).wait()

    def _zero_body(i, carry):
        idx16 = idx_f[pl.ds(i * 16, 16)]
        for l in range(16):
            @pl.when(idx16[l] == 0)
            def _():
                r = i * 16 + l
                for off in (0, 16, 32, 34):
                    rows_v[r, pl.ds(off, 16)] = zeros16
        return carry
    lax.fori_loop(0, ROWS_W // 16, _zero_body, 0)
    pltpu.sync_copy(rows_v, emb_out.at[pl.ds(wid * ROWS_W, ROWS_W)])

    def _hz(i, carry):
        for u in range(8):
            hist[pl.ds(i * 128 + u * 16, 16)] = zeros16
        return carry
    lax.fori_loop(0, (16 * HBIN) // 128, _hz, 0)

    def _hz2(i, carry):
        for u in range(8):
            hist2[pl.ds(i * 128 + u * 16, 16)] = zeros16
        return carry
    lax.fori_loop(0, NB2 // 128, _hz2, 0)

    pltpu.sync_copy(src_flat.at[pl.ds(wid * EW, EW)], sv)
    pltpu.sync_copy(dst_flat.at[pl.ds(wid * EW, EW)], dv)

    def _acc(i, carry):
        s16 = sv[pl.ds(i * 16, 16)]
        d16 = dv[pl.ds(i * 16, 16)]
        k = lane * HBIN + d16 * S2 + s16
        plsc.addupdate_scatter(hist, [k], ones16, mask=lane < 16)
        return carry
    lax.fori_loop(0, EW // 16, _acc, 0)

    def _red(c, carry):
        base = c * 16
        acc = hist[pl.ds(base, 16)]
        for l in range(1, 16):
            acc = acc + hist[pl.ds(l * HBIN + base, 16)]
        j = base + lane
        bvec = j // S2
        svec = j - bvec * S2
        plsc.store_scatter(hist2, [svec * B + bvec], acc)
        return carry
    lax.fori_loop(0, HBIN // 16, _red, 0)
    pltpu.sync_copy(hist2, cnt_out.at[pl.ds(wid * NB2, NB2)])


@functools.cache
def _make_sc_call():
  return pl.kernel(
    _sc_body,
    out_type=(
        jax.ShapeDtypeStruct((T * B, D), jnp.float32),
        jax.ShapeDtypeStruct((NW * NB2,), jnp.float32),
    ),
    mesh=plsc.VectorSubcoreMesh(core_axis_name="c", subcore_axis_name="s",
                                num_cores=NC, num_subcores=NS),
    compiler_params=pltpu.CompilerParams(needs_layout_passes=False,
                                         use_tc_tiling_on_sc=False),
    scratch_types=[
        pltpu.VMEM((ROWS_W,), jnp.int32),
        pltpu.VMEM((ROWS_W, D), jnp.float32),
        pltpu.VMEM((EW,), jnp.int32),
        pltpu.VMEM((EW,), jnp.int32),
        pltpu.VMEM((16 * HBIN,), jnp.float32),
        pltpu.VMEM((NB2,), jnp.float32),
        pltpu.SemaphoreType.DMA,
    ],
  )


def _gather8(tbl_any, idx_smem, buf, sem, mask_zero):
    f32 = jnp.float32
    i32 = jnp.int32
    rows = []
    riota = lax.broadcasted_iota(i32, (8, 1), 0)
    for i in range(S2):
        si = idx_smem[i]
        a = pl.multiple_of((si // 8) * 8, 8)
        cp = pltpu.make_async_copy(tbl_any.at[pl.ds(a, 8), :], buf, sem)
        cp.start()
        cp.wait()
        sel = (riota == si - a).astype(f32)
        row = jnp.sum(buf[:] * sel, axis=0, keepdims=True)
        if mask_zero:
            row = row * (si != 0).astype(f32)
        rows.append(row)
    rows.append(jnp.zeros((8 - S2, D), f32))
    return jnp.concatenate(rows, axis=0)


def _sr_compute(emb, cnts, ur, ir, Wx, Wh, bg, W1a, W1b, f2w, f2b,
                W2a, W2b):
    f32 = jnp.float32
    bias = bg[0:1, :]

    def step(t, hc):
        h, c = hc
        x = emb[pl.ds(t * B, B), :]
        g = (jnp.dot(x, Wx[:], preferred_element_type=f32)
             + jnp.dot(h, Wh[:], preferred_element_type=f32) + bias)
        i = jax.nn.sigmoid(g[:, 0:128])
        f = jax.nn.sigmoid(g[:, 128:256])
        gg = jnp.tanh(g[:, 256:384])
        o = jax.nn.sigmoid(g[:, 384:512])
        c = f * c + i * gg
        h = o * jnp.tanh(c)
        return (h, c)

    h0 = jnp.zeros((B, 128), f32)
    h, _ = lax.fori_loop(0, T, step, (h0, h0))

    g5 = jnp.dot(ir, Wx[:], preferred_element_type=f32) + bias
    c5 = jax.nn.sigmoid(g5[:, 0:128]) * jnp.tanh(g5[:, 256:384])
    h5 = jax.nn.sigmoid(g5[:, 384:512]) * jnp.tanh(c5)

    ls2 = jnp.maximum(jnp.dot(ur, W1a[:], preferred_element_type=f32)
                      + jnp.dot(h5, W1b[:], preferred_element_type=f32), 0.0)
    rowm = (lax.broadcasted_iota(jnp.int32, (8, 1), 0) < S2).astype(f32)
    ls2 = ls2 * rowm

    cbs = jnp.sum(cnts[:], axis=0).T
    P = lax.dot_general(h, ls2, (((1,), (1,)), ((), ())),
                        preferred_element_type=f32)
    mx = jnp.max(P, axis=1, keepdims=True)
    wt = cbs * jnp.exp(P - mx)
    den = jnp.sum(wt, axis=1, keepdims=True)
    wn = wt / jnp.where(den == 0.0, 1.0, den)
    rst = jnp.dot(wn, ls2, preferred_element_type=f32)
    feat = jnp.maximum(jnp.dot(rst, f2w[:], preferred_element_type=f32)
                       + f2b[0:1, :], 0.0)
    return (jnp.dot(h, W2a[:], preferred_element_type=f32)
            + jnp.dot(feat, W2b[:], preferred_element_type=f32))


def _a_body(emb, cnts, iu_s, ii_s, user_any, item_any, Wx, Wh, bg,
            W1a, W1b, f2w, f2b, W2a, W2b, sr_out, g8_buf, g8_sem):
    ur = _gather8(user_any, iu_s, g8_buf, g8_sem, mask_zero=False)
    ir = _gather8(item_any, ii_s, g8_buf, g8_sem, mask_zero=True)
    sr_out[:] = _sr_compute(emb, cnts, ur, ir, Wx, Wh, bg,
                            W1a, W1b, f2w, f2b, W2a, W2b)


def _c_body(sr_ref, ita_ref, itb_ref, out_ref):
    it = jnp.concatenate([ita_ref[1:], itb_ref[0:1]], axis=0)
    out_ref[:] = lax.dot_general(sr_ref[:], it, (((1,), (1,)), ((), ())),
                                 preferred_element_type=jnp.float32)


def kernel(input_session, support_nodes_layer1, support_nodes_layer2,
           support_sessions_layer1, support_sessions_layer2,
           edge_index1, edge_index2,
           user_emb, item_emb, Wih, Whh, bih, bhh,
           W1, fc1_w, fc1_b, fc2_w, fc2_b, W2):
    f32 = jnp.float32
    i32 = jnp.int32

    idx_flat = input_session.T.reshape(-1).astype(i32)
    src_flat = edge_index2[0].astype(i32)
    dst_flat = edge_index2[1].astype(i32)
    iu8 = jnp.zeros((8,), i32).at[:S2].set(support_nodes_layer2.astype(i32))
    ii8 = jnp.zeros((8,), i32).at[:S2].set(
        support_sessions_layer2[:, 0].astype(i32))

    def padto(a, r, c):
        return jnp.zeros((r, c), f32).at[:a.shape[0], :a.shape[1]].set(a)

    WihT, WhhT = Wih.T, Whh.T
    Wx = jnp.zeros((D, 512), f32)
    Wh = jnp.zeros((128, 512), f32)
    bg = jnp.zeros((8, 512), f32)
    bsum = bih + bhh
    for g in range(4):
        Wx = Wx.at[:, g * 128:g * 128 + D].set(WihT[:, g * D:(g + 1) * D])
        Wh = Wh.at[:D, g * 128:g * 128 + D].set(WhhT[:, g * D:(g + 1) * D])
        bg = bg.at[:, g * 128:g * 128 + D].set(
            jnp.broadcast_to(bsum[g * D:(g + 1) * D], (8, D)))
    W1a = padto(W1[:, :D].T, D, 128)
    W1b = padto(W1[:, D:].T, 128, 128)
    f2w = padto(fc2_w.T, 128, 128)
    f2b = jnp.zeros((8, 128), f32).at[:, :D].set(jnp.broadcast_to(fc2_b, (8, D)))
    W2a = padto(W2[:, :D].T, 128, D)
    W2b = padto(W2[:, D:].T, 128, D)

    emb, cnts = _make_sc_call()(item_emb, idx_flat, src_flat, dst_flat)
    cnts3 = cnts.reshape(NW, 8, B)

    vspec = pl.BlockSpec(memory_space=pltpu.VMEM)
    sr = pl.pallas_call(
        _a_body,
        in_specs=[vspec, vspec,
                  pl.BlockSpec(memory_space=pltpu.SMEM),
                  pl.BlockSpec(memory_space=pltpu.SMEM),
                  pl.BlockSpec(memory_space=pl.ANY),
                  pl.BlockSpec(memory_space=pl.ANY),
                  vspec, vspec, vspec, vspec, vspec, vspec, vspec, vspec, vspec],
        out_shape=jax.ShapeDtypeStruct((B, D), f32),
        scratch_shapes=[pltpu.VMEM((8, D), f32), pltpu.SemaphoreType.DMA],
    )(emb, cnts3, iu8, ii8, user_emb, item_emb, Wx, Wh, bg,
      W1a, W1b, f2w, f2b, W2a, W2b)

    nblk = (NV + CB - 1) // CB
    nb8 = item_emb.shape[0] // 8 - 1
    logits = pl.pallas_call(
        _c_body,
        grid=(nblk,),
        in_specs=[pl.BlockSpec((B, D), lambda j: (0, 0)),
                  pl.BlockSpec((CB, D), lambda j: (j, 0)),
                  pl.BlockSpec((8, D),
                               lambda j: (jnp.minimum((j + 1) * (CB // 8), nb8), 0))],
        out_specs=pl.BlockSpec((B, CB), lambda j: (0, j)),
        out_shape=jax.ShapeDtypeStruct((B, NV), f32),
    )(sr, item_emb, item_emb)
    return logits
